# baseline (speedup 1.0000x reference)
import jax
import jax.numpy as jnp
from jax import lax
from jax.experimental import pallas as pl
from jax.experimental.pallas import tpu as pltpu
from jax.experimental.pallas import tpu_sc as plsc

_NC = 2
_NS = 16
_NW = _NC * _NS
_G = 128
_J = 8
_CB = _J * _G
_D = 32
_TILE = 8 * _G
_CHW = 4 * _J * _TILE


def _gather_body(idx_hbm, table_hbm, out_hbm,
                 idx0, idx1, rows0, rows1, tb,
                 isem0, isem1, gsem0, gsem1, osem):
    wid = lax.axis_index("s") * _NC + lax.axis_index("c")
    total_chunks = idx_hbm.shape[0] // _J
    chunks_per_w = total_chunks // _NW
    cps = 16384 // _CB
    c0 = wid * chunks_per_w

    idx_b = (idx0, idx1)
    rows_b = (rows0, rows1)
    isem = (isem0, isem1)
    gsem = (gsem0, gsem1)

    iota = lax.iota(jnp.int32, 16)
    addr_lo = ((iota >> 3) << 13) + ((iota & 7) << 7)
    addr_hi = addr_lo + (2 << 13)

    def idx_load(g, b):
        c = jnp.minimum(c0 + g, c0 + chunks_per_w - 1)
        pltpu.async_copy(idx_hbm.at[pl.ds(c * _J, _J)], idx_b[b], isem[b])

    def wait_idx(b):
        pltpu.make_async_copy(idx_hbm.at[pl.ds(0, _J)],
                              idx_b[b], isem[b]).wait()

    def fire_gathers(b):
        for j in range(_J):
            pltpu.async_copy(table_hbm.at[idx_b[b].at[j]],
                             rows_b[b].at[pl.ds(j * _G, _G)], gsem[b])

    def wait_gathers(b):
        pltpu.make_async_copy(table_hbm.at[pl.ds(0, _CB)],
                              rows_b[b], gsem[b]).wait()

    def transpose(b):
        rows = rows_b[b]

        @pl.loop(0, _CB, unroll=8)
        def _(i):
            off = ((i >> 7) << 10) + (i & 127)
            va = rows[i, pl.ds(0, 16)]
            vb = rows[i, pl.ds(16, 16)]
            plsc.store_scatter(tb, [addr_lo + off], va)
            plsc.store_scatter(tb, [addr_hi + off], vb)

    def stores(g):
        c = c0 + g
        s = c // cps
        tt = c % cps
        for q in range(4):
            dst = (s * 4 + q) * (128 * _TILE) + tt * (_J * _TILE)
            pltpu.async_copy(tb.at[pl.ds(q * _J * _TILE, _J * _TILE)],
                             out_hbm.at[pl.ds(dst, _J * _TILE)], osem)

    def wait_stores():
        for _q in range(4):
            pltpu.make_async_copy(tb.at[pl.ds(0, _J * _TILE)],
                                  out_hbm.at[pl.ds(0, _J * _TILE)],
                                  osem).wait()

    idx_load(0, 0)
    wait_idx(0)
    fire_gathers(0)
    idx_load(1, 1)
    wait_idx(1)
    fire_gathers(1)
    wait_gathers(0)
    transpose(0)
    stores(0)
    idx_load(2, 0)

    @pl.loop(2, chunks_per_w, step=2)
    def _(t):
        for b in range(2):
            g = t + b
            wait_idx(b)
            fire_gathers(b)
            wait_gathers(1 - b)
            wait_stores()
            transpose(1 - b)
            stores(g - 1)
            idx_load(g + 1, 1 - b)

    last_b = (chunks_per_w - 1) % 2
    wait_gathers(last_b)
    wait_stores()
    transpose(last_b)
    stores(chunks_per_w - 1)
    wait_stores()
    wait_idx(1 - last_b)


@jax.jit
def _embedding_lookup(idxf, weight):
    n_rows = idxf.shape[0]
    run = pl.kernel(
        _gather_body,
        out_type=jax.ShapeDtypeStruct((n_rows * _G * _D,), jnp.float32),
        mesh=plsc.VectorSubcoreMesh(
            core_axis_name="c", subcore_axis_name="s",
            num_cores=_NC, num_subcores=_NS),
        scratch_types=[
            pltpu.VMEM((_J, _G), jnp.int32),
            pltpu.VMEM((_J, _G), jnp.int32),
            pltpu.VMEM((_CB, _D), jnp.float32),
            pltpu.VMEM((_CB, _D), jnp.float32),
            pltpu.VMEM((_CHW,), jnp.float32),
            pltpu.SemaphoreType.DMA,
            pltpu.SemaphoreType.DMA,
            pltpu.SemaphoreType.DMA,
            pltpu.SemaphoreType.DMA,
            pltpu.SemaphoreType.DMA,
        ],
        compiler_params=pltpu.CompilerParams(use_tc_tiling_on_sc=False,
                                             needs_layout_passes=False),
    )
    return run(idxf, weight)


def kernel(x, weight):
    b, s = x.shape
    d = weight.shape[1]
    idxf = x.T.astype(jnp.int32).reshape(-1, _G)
    out_flat = _embedding_lookup(idxf, weight)
    nbt = b // _G
    ngq = d // 8
    t = out_flat.reshape(s, ngq, nbt, 8, _G)
    return t.transpose(2, 4, 0, 1, 3).reshape(b, s, d)


# speedup vs baseline: 1.1439x; 1.1439x over previous
import jax
import jax.numpy as jnp
from jax import lax
from jax.experimental import pallas as pl
from jax.experimental.pallas import tpu as pltpu
from jax.experimental.pallas import tpu_sc as plsc

_NC = 2
_NS = 16
_NW = _NC * _NS
_G = 128
_J = 8
_CB = _J * _G
_D = 32
_TILE = 8 * _G
_CHW = 4 * _J * _TILE


def _gather_body(idx_hbm, table_hbm, out_hbm,
                 idx0, idx1, rows0, rows1, tb,
                 isem0, isem1, gsem0, gsem1, osem):
    wid = lax.axis_index("s") * _NC + lax.axis_index("c")
    total_chunks = idx_hbm.shape[0] // _J
    chunks_per_w = total_chunks // _NW
    cps = 16384 // _CB
    c0 = wid * chunks_per_w

    idx_b = (idx0, idx1)
    rows_b = (rows0, rows1)
    isem = (isem0, isem1)
    gsem = (gsem0, gsem1)

    iota = lax.iota(jnp.int32, 16)
    addr_lo = ((iota >> 3) << 13) + ((iota & 7) << 7)
    addr_hi = addr_lo + (2 << 13)

    def idx_load(g, b):
        c = jnp.minimum(c0 + g, c0 + chunks_per_w - 1)
        pltpu.async_copy(idx_hbm.at[pl.ds(c * _J, _J)], idx_b[b], isem[b])

    def wait_idx(b):
        pltpu.make_async_copy(idx_hbm.at[pl.ds(0, _J)],
                              idx_b[b], isem[b]).wait()

    def fire_gathers(b):
        for j in range(_J):
            pltpu.async_copy(table_hbm.at[idx_b[b].at[j]],
                             rows_b[b].at[pl.ds(j * _G, _G)], gsem[b])

    def wait_gathers(b):
        pltpu.make_async_copy(table_hbm.at[pl.ds(0, _CB)],
                              rows_b[b], gsem[b]).wait()

    def transpose(b):
        rows = rows_b[b]

        for tj in range(_J):
            @plsc.parallel_loop(tj * _G, (tj + 1) * _G, unroll=8)
            def _(i):
                off = (tj * 1024 - tj * _G) + i
                va = rows[i, pl.ds(0, 16)]
                vb = rows[i, pl.ds(16, 16)]
                plsc.store_scatter(tb, [addr_lo + off], va)
                plsc.store_scatter(tb, [addr_hi + off], vb)

    def stores(g):
        c = c0 + g
        s = c // cps
        tt = c % cps
        for q in range(4):
            dst = (s * 4 + q) * (128 * _TILE) + tt * (_J * _TILE)
            pltpu.async_copy(tb.at[pl.ds(q * _J * _TILE, _J * _TILE)],
                             out_hbm.at[pl.ds(dst, _J * _TILE)], osem)

    def wait_stores():
        for _q in range(4):
            pltpu.make_async_copy(tb.at[pl.ds(0, _J * _TILE)],
                                  out_hbm.at[pl.ds(0, _J * _TILE)],
                                  osem).wait()

    idx_load(0, 0)
    wait_idx(0)
    fire_gathers(0)
    idx_load(1, 1)
    wait_idx(1)
    fire_gathers(1)
    wait_gathers(0)
    transpose(0)
    stores(0)
    idx_load(2, 0)

    @pl.loop(2, chunks_per_w, step=2)
    def _(t):
        for b in range(2):
            g = t + b
            wait_idx(b)
            fire_gathers(b)
            wait_gathers(1 - b)
            wait_stores()
            transpose(1 - b)
            stores(g - 1)
            idx_load(g + 1, 1 - b)

    last_b = (chunks_per_w - 1) % 2
    wait_gathers(last_b)
    wait_stores()
    transpose(last_b)
    stores(chunks_per_w - 1)
    wait_stores()
    wait_idx(1 - last_b)


@jax.jit
def _embedding_lookup(idxf, weight):
    n_rows = idxf.shape[0]
    run = pl.kernel(
        _gather_body,
        out_type=jax.ShapeDtypeStruct((n_rows * _G * _D,), jnp.float32),
        mesh=plsc.VectorSubcoreMesh(
            core_axis_name="c", subcore_axis_name="s",
            num_cores=_NC, num_subcores=_NS),
        scratch_types=[
            pltpu.VMEM((_J, _G), jnp.int32),
            pltpu.VMEM((_J, _G), jnp.int32),
            pltpu.VMEM((_CB, _D), jnp.float32),
            pltpu.VMEM((_CB, _D), jnp.float32),
            pltpu.VMEM((_CHW,), jnp.float32),
            pltpu.SemaphoreType.DMA,
            pltpu.SemaphoreType.DMA,
            pltpu.SemaphoreType.DMA,
            pltpu.SemaphoreType.DMA,
            pltpu.SemaphoreType.DMA,
        ],
        compiler_params=pltpu.CompilerParams(use_tc_tiling_on_sc=False,
                                             needs_layout_passes=False),
    )
    return run(idxf, weight)


def kernel(x, weight):
    b, s = x.shape
    d = weight.shape[1]
    idxf = x.T.astype(jnp.int32).reshape(-1, _G)
    out_flat = _embedding_lookup(idxf, weight)
    nbt = b // _G
    ngq = d // 8
    t = out_flat.reshape(s, ngq, nbt, 8, _G)
    return t.transpose(2, 4, 0, 1, 3).reshape(b, s, d)


# speedup vs baseline: 2.9856x; 2.6100x over previous
import jax
import jax.numpy as jnp
from jax import lax
from jax.experimental import pallas as pl
from jax.experimental.pallas import tpu as pltpu
from jax.experimental.pallas import tpu_sc as plsc

_NC = 2
_NS = 16
_NW = _NC * _NS
_G = 128
_J = 8
_CB = _J * _G
_D = 32
_TILE = 8 * _G
_QS = 136
_RS = 4 * _QS + 1


def _gather_body(idx_hbm, table_hbm, out_hbm,
                 idx0, idx1, rows0, rows1, tb,
                 isem0, isem1, gsem0, gsem1, osem):
    wid = lax.axis_index("s") * _NC + lax.axis_index("c")
    total_chunks = idx_hbm.shape[0] // _J
    chunks_per_w = total_chunks // _NW
    cps = 16384 // _CB
    c0 = wid * chunks_per_w

    idx_b = (idx0, idx1)
    rows_b = (rows0, rows1)
    isem = (isem0, isem1)
    gsem = (gsem0, gsem1)

    iota = lax.iota(jnp.int32, 16)
    row_base = iota & 7
    col_lo = (iota >> 3) * _QS
    col_hi = col_lo + 2 * _QS

    def idx_load(g, b):
        c = jnp.minimum(c0 + g, c0 + chunks_per_w - 1)
        pltpu.async_copy(idx_hbm.at[pl.ds(c * _J, _J)], idx_b[b], isem[b])

    def wait_idx(b):
        pltpu.make_async_copy(idx_hbm.at[pl.ds(0, _J)],
                              idx_b[b], isem[b]).wait()

    def fire_gathers(b):
        for j in range(_J):
            pltpu.async_copy(table_hbm.at[idx_b[b].at[j]],
                             rows_b[b].at[pl.ds(j * _G, _G)], gsem[b])

    def wait_gathers(b):
        pltpu.make_async_copy(table_hbm.at[pl.ds(0, _CB)],
                              rows_b[b], gsem[b]).wait()

    def transpose(b):
        rows = rows_b[b]

        for tj in range(_J):
            rvec = row_base + tj * 8

            @plsc.parallel_loop(tj * _G, (tj + 1) * _G, unroll=8)
            def _(i):
                c = i - tj * _G
                va = rows[i, pl.ds(0, 16)]
                vb = rows[i, pl.ds(16, 16)]
                plsc.store_scatter(tb, [rvec, col_lo + c], va)
                plsc.store_scatter(tb, [rvec, col_hi + c], vb)

    def stores(g):
        c = c0 + g
        s = c // cps
        tt = c % cps
        for q in range(4):
            r0 = ((s * 4 + q) * 128 + tt * _J) * 8
            pltpu.async_copy(tb.at[pl.ds(0, 8 * _J), pl.ds(q * _QS, _G)],
                             out_hbm.at[pl.ds(r0, 8 * _J)], osem)

    def wait_stores():
        for _q in range(4):
            pltpu.make_async_copy(tb.at[pl.ds(0, 8 * _J), pl.ds(0, _G)],
                                  out_hbm.at[pl.ds(0, 8 * _J)],
                                  osem).wait()

    idx_load(0, 0)
    wait_idx(0)
    fire_gathers(0)
    idx_load(1, 1)
    wait_idx(1)
    fire_gathers(1)
    wait_gathers(0)
    transpose(0)
    stores(0)
    idx_load(2, 0)

    @pl.loop(2, chunks_per_w, step=2)
    def _(t):
        for b in range(2):
            g = t + b
            wait_idx(b)
            fire_gathers(b)
            wait_gathers(1 - b)
            wait_stores()
            transpose(1 - b)
            stores(g - 1)
            idx_load(g + 1, 1 - b)

    last_b = (chunks_per_w - 1) % 2
    wait_gathers(last_b)
    wait_stores()
    transpose(last_b)
    stores(chunks_per_w - 1)
    wait_stores()
    wait_idx(1 - last_b)


@jax.jit
def _embedding_lookup(idxf, weight):
    n_rows = idxf.shape[0]
    run = pl.kernel(
        _gather_body,
        out_type=jax.ShapeDtypeStruct((n_rows * _G * _D // _G, _G),
                                      jnp.float32),
        mesh=plsc.VectorSubcoreMesh(
            core_axis_name="c", subcore_axis_name="s",
            num_cores=_NC, num_subcores=_NS),
        scratch_types=[
            pltpu.VMEM((_J, _G), jnp.int32),
            pltpu.VMEM((_J, _G), jnp.int32),
            pltpu.VMEM((_CB, _D), jnp.float32),
            pltpu.VMEM((_CB, _D), jnp.float32),
            pltpu.VMEM((8 * _J, _RS), jnp.float32),
            pltpu.SemaphoreType.DMA,
            pltpu.SemaphoreType.DMA,
            pltpu.SemaphoreType.DMA,
            pltpu.SemaphoreType.DMA,
            pltpu.SemaphoreType.DMA,
        ],
        compiler_params=pltpu.CompilerParams(use_tc_tiling_on_sc=False,
                                             needs_layout_passes=False),
    )
    return run(idxf, weight)


def kernel(x, weight):
    b, s = x.shape
    d = weight.shape[1]
    idxf = x.T.astype(jnp.int32).reshape(-1, _G)
    out_flat = _embedding_lookup(idxf, weight)
    nbt = b // _G
    ngq = d // 8
    t = out_flat.reshape(s, ngq, nbt, 8, _G)
    return t.transpose(2, 4, 0, 1, 3).reshape(b, s, d)


# speedup vs baseline: 3.0098x; 1.0081x over previous
import jax
import jax.numpy as jnp
from jax import lax
from jax.experimental import pallas as pl
from jax.experimental.pallas import tpu as pltpu
from jax.experimental.pallas import tpu_sc as plsc

_NC = 2
_NS = 16
_NW = _NC * _NS
_G = 128
_J = 8
_CB = _J * _G
_D = 32
_TILE = 8 * _G
_QS = 136
_RS = 4 * _QS + 1


def _gather_body(idx_hbm, table_hbm, out_hbm,
                 idx0, idx1, rows0, rows1, tb,
                 isem0, isem1, gsem0, gsem1, osem):
    wid = lax.axis_index("s") * _NC + lax.axis_index("c")
    nb = idx_hbm.shape[1] * idx_hbm.shape[3]
    cps = nb // _CB
    total_chunks = idx_hbm.shape[0] * 8 * cps
    chunks_per_w = total_chunks // _NW
    c0 = wid * chunks_per_w

    idx_b = (idx0, idx1)
    rows_b = (rows0, rows1)
    isem = (isem0, isem1)
    gsem = (gsem0, gsem1)

    iota = lax.iota(jnp.int32, 16)
    row_base = iota & 7
    col_lo = (iota >> 3) * _QS
    col_hi = col_lo + 2 * _QS

    def idx_load(g, b):
        c = jnp.minimum(c0 + g, c0 + chunks_per_w - 1)
        s = c // cps
        bi0 = (c % cps) * _J
        pltpu.async_copy(
            idx_hbm.at[pl.ds(s // 8, 1), pl.ds(bi0, _J), pl.ds(s % 8, 1)],
            idx_b[b], isem[b])

    def wait_idx(b):
        pltpu.make_async_copy(
            idx_hbm.at[pl.ds(0, 1), pl.ds(0, _J), pl.ds(0, 1)],
            idx_b[b], isem[b]).wait()

    def fire_gathers(b):
        for j in range(_J):
            pltpu.async_copy(table_hbm.at[idx_b[b].at[0, j, 0]],
                             rows_b[b].at[pl.ds(j * _G, _G)], gsem[b])

    def wait_gathers(b):
        pltpu.make_async_copy(table_hbm.at[pl.ds(0, _CB)],
                              rows_b[b], gsem[b]).wait()

    def transpose(b):
        rows = rows_b[b]

        for tj in range(_J):
            rvec = row_base + tj * 8

            @plsc.parallel_loop(tj * _G, (tj + 1) * _G, unroll=8)
            def _(i):
                c = i - tj * _G
                va = rows[i, pl.ds(0, 16)]
                vb = rows[i, pl.ds(16, 16)]
                plsc.store_scatter(tb, [rvec, col_lo + c], va)
                plsc.store_scatter(tb, [rvec, col_hi + c], vb)

    def stores(g):
        c = c0 + g
        s = c // cps
        tt = c % cps
        for q in range(4):
            r0 = ((s * 4 + q) * 128 + tt * _J) * 8
            pltpu.async_copy(tb.at[pl.ds(0, 8 * _J), pl.ds(q * _QS, _G)],
                             out_hbm.at[pl.ds(r0, 8 * _J)], osem)

    def wait_stores():
        for _q in range(4):
            pltpu.make_async_copy(tb.at[pl.ds(0, 8 * _J), pl.ds(0, _G)],
                                  out_hbm.at[pl.ds(0, 8 * _J)],
                                  osem).wait()

    idx_load(0, 0)
    wait_idx(0)
    fire_gathers(0)
    idx_load(1, 1)
    wait_idx(1)
    fire_gathers(1)
    wait_gathers(0)
    transpose(0)
    stores(0)
    idx_load(2, 0)

    @pl.loop(2, chunks_per_w, step=2)
    def _(t):
        for b in range(2):
            g = t + b
            wait_idx(b)
            fire_gathers(b)
            wait_gathers(1 - b)
            wait_stores()
            transpose(1 - b)
            stores(g - 1)
            idx_load(g + 1, 1 - b)

    last_b = (chunks_per_w - 1) % 2
    wait_gathers(last_b)
    wait_stores()
    transpose(last_b)
    stores(chunks_per_w - 1)
    wait_stores()
    wait_idx(1 - last_b)


@jax.jit
def _embedding_lookup(idx4, weight):
    n_idx = idx4.shape[0] * idx4.shape[1] * idx4.shape[2] * idx4.shape[3]
    run = pl.kernel(
        _gather_body,
        out_type=jax.ShapeDtypeStruct((n_idx * _D // _G, _G),
                                      jnp.float32),
        mesh=plsc.VectorSubcoreMesh(
            core_axis_name="c", subcore_axis_name="s",
            num_cores=_NC, num_subcores=_NS),
        scratch_types=[
            pltpu.VMEM((1, _J, 1, _G), jnp.int32),
            pltpu.VMEM((1, _J, 1, _G), jnp.int32),
            pltpu.VMEM((_CB, _D), jnp.float32),
            pltpu.VMEM((_CB, _D), jnp.float32),
            pltpu.VMEM((8 * _J, _RS), jnp.float32),
            pltpu.SemaphoreType.DMA,
            pltpu.SemaphoreType.DMA,
            pltpu.SemaphoreType.DMA,
            pltpu.SemaphoreType.DMA,
            pltpu.SemaphoreType.DMA,
        ],
        compiler_params=pltpu.CompilerParams(use_tc_tiling_on_sc=False,
                                             needs_layout_passes=False),
    )
    return run(idx4, weight)


def kernel(x, weight):
    b, s = x.shape
    d = weight.shape[1]
    xt = x.T.astype(jnp.int32)
    idx4 = xt.reshape(s // 8, 8, b // _G, _G).transpose(0, 2, 1, 3)
    out_flat = _embedding_lookup(idx4, weight)
    nbt = b // _G
    ngq = d // 8
    t = out_flat.reshape(s, ngq, nbt, 8, _G)
    return t.transpose(2, 4, 0, 1, 3).reshape(b, s, d)


# speedup vs baseline: 3.2153x; 1.0683x over previous
import jax
import jax.numpy as jnp
from jax import lax
from jax.experimental import pallas as pl
from jax.experimental.pallas import tpu as pltpu
from jax.experimental.pallas import tpu_sc as plsc

_NC = 2
_NS = 16
_NW = _NC * _NS
_G = 128
_J = 8
_CB = _J * _G
_D = 32
_TILE = 8 * _G
_QS = 136
_RS = 4 * _QS + 1


def _gather_body(idx_hbm, table_hbm, out_hbm,
                 idx0, idx1, rows0, rows1, tb,
                 isem0, isem1, gsem0, gsem1, osem):
    wid = lax.axis_index("s") * _NC + lax.axis_index("c")
    nb = idx_hbm.shape[1] * idx_hbm.shape[3]
    cps = nb // _CB
    total_chunks = idx_hbm.shape[0] * 8 * cps
    chunks_per_w = total_chunks // _NW
    c0 = wid * chunks_per_w

    idx_b = (idx0, idx1)
    rows_b = (rows0, rows1)
    isem = (isem0, isem1)
    gsem = (gsem0, gsem1)

    iota = lax.iota(jnp.int32, 16)
    row_base = iota & 7
    col_lo = (iota >> 3) * _QS
    col_hi = col_lo + 2 * _QS

    def idx_load(g, b):
        c = jnp.minimum(c0 + g, c0 + chunks_per_w - 1)
        s = c // cps
        bi0 = (c % cps) * _J
        pltpu.async_copy(
            idx_hbm.at[pl.ds(s // 8, 1), pl.ds(bi0, _J), pl.ds(s % 8, 1)],
            idx_b[b], isem[b])

    def wait_idx(b):
        pltpu.make_async_copy(
            idx_hbm.at[pl.ds(0, 1), pl.ds(0, _J), pl.ds(0, 1)],
            idx_b[b], isem[b]).wait()

    def fire_gathers(b):
        for j in range(_J):
            pltpu.async_copy(table_hbm.at[idx_b[b].at[0, j, 0]],
                             rows_b[b].at[pl.ds(j * _G, _G)], gsem[b])

    def wait_gathers(b):
        pltpu.make_async_copy(table_hbm.at[pl.ds(0, _CB)],
                              rows_b[b], gsem[b]).wait()

    def transpose(b):
        rows = rows_b[b]

        for tj in range(_J):
            rvec = row_base + tj * 8

            @plsc.parallel_loop(tj * _G, (tj + 1) * _G, unroll=8)
            def _(i):
                c = i - tj * _G
                va = rows[i, pl.ds(0, 16)]
                vb = rows[i, pl.ds(16, 16)]
                plsc.store_scatter(tb, [rvec, col_lo + c], va)
                plsc.store_scatter(tb, [rvec, col_hi + c], vb)

    def stores(g):
        c = c0 + g
        s = c // cps
        tt = c % cps
        for q in range(4):
            r0 = ((s * 4 + q) * 128 + tt * _J) * 8
            pltpu.async_copy(tb.at[pl.ds(0, 8 * _J), pl.ds(q * _QS, _G)],
                             out_hbm.at[pl.ds(r0, 8 * _J)], osem)

    def wait_stores():
        for _q in range(4):
            pltpu.make_async_copy(tb.at[pl.ds(0, 8 * _J), pl.ds(0, _G)],
                                  out_hbm.at[pl.ds(0, 8 * _J)],
                                  osem).wait()

    idx_load(0, 0)
    wait_idx(0)
    fire_gathers(0)
    idx_load(1, 1)
    wait_idx(1)
    fire_gathers(1)
    wait_gathers(0)
    transpose(0)
    stores(0)
    idx_load(2, 0)

    @pl.loop(2, chunks_per_w, step=2)
    def _(t):
        for b in range(2):
            g = t + b
            wait_idx(b)
            fire_gathers(b)
            wait_gathers(1 - b)
            idx_load(g + 1, 1 - b)
            wait_stores()
            transpose(1 - b)
            stores(g - 1)

    last_b = (chunks_per_w - 1) % 2
    wait_gathers(last_b)
    wait_stores()
    transpose(last_b)
    stores(chunks_per_w - 1)
    wait_stores()
    wait_idx(1 - last_b)


@jax.jit
def _embedding_lookup(idx4, weight):
    n_idx = idx4.shape[0] * idx4.shape[1] * idx4.shape[2] * idx4.shape[3]
    run = pl.kernel(
        _gather_body,
        out_type=jax.ShapeDtypeStruct((n_idx * _D // _G, _G),
                                      jnp.float32),
        mesh=plsc.VectorSubcoreMesh(
            core_axis_name="c", subcore_axis_name="s",
            num_cores=_NC, num_subcores=_NS),
        scratch_types=[
            pltpu.VMEM((1, _J, 1, _G), jnp.int32),
            pltpu.VMEM((1, _J, 1, _G), jnp.int32),
            pltpu.VMEM((_CB, _D), jnp.float32),
            pltpu.VMEM((_CB, _D), jnp.float32),
            pltpu.VMEM((8 * _J, _RS), jnp.float32),
            pltpu.SemaphoreType.DMA,
            pltpu.SemaphoreType.DMA,
            pltpu.SemaphoreType.DMA,
            pltpu.SemaphoreType.DMA,
            pltpu.SemaphoreType.DMA,
        ],
        compiler_params=pltpu.CompilerParams(use_tc_tiling_on_sc=False,
                                             needs_layout_passes=False),
    )
    return run(idx4, weight)


def kernel(x, weight):
    b, s = x.shape
    d = weight.shape[1]
    xt = x.T.astype(jnp.int32)
    idx4 = xt.reshape(s // 8, 8, b // _G, _G).transpose(0, 2, 1, 3)
    out_flat = _embedding_lookup(idx4, weight)
    nbt = b // _G
    ngq = d // 8
    t = out_flat.reshape(s, ngq, nbt, 8, _G)
    return t.transpose(2, 4, 0, 1, 3).reshape(b, s, d)
